# baseline (speedup 1.0000x reference)
import jax
import jax.numpy as jnp
from jax import lax
from jax.experimental import pallas as pl
from jax.experimental.pallas import tpu as pltpu
from jax.experimental.pallas import tpu_sc as plsc

_NC = 2
_NS = 16
_L = 16
_NW = _NC * _NS

_NB = 64


def _sc_lorentz_products(t32, idx_i, ks3d, B, N, D):
    bpw = B // _NW
    nsub = bpw // _NB
    rows = _NB * N
    ngrp = rows // _L
    krows = rows // 128
    kpw = bpw * N // 128

    mesh = plsc.VectorSubcoreMesh(core_axis_name="c", subcore_axis_name="s")

    @pl.kernel(
        out_type=jax.ShapeDtypeStruct((B * N,), jnp.float32),
        mesh=mesh,
        scratch_types=[
            pltpu.VMEM((bpw,), jnp.int32),
            pltpu.VMEM((kpw, 128), jnp.int32),
            pltpu.VMEM((2, _NB, D), jnp.float32),
            pltpu.VMEM((2, rows, D), jnp.float32),
            pltpu.VMEM((2, rows), jnp.float32),
            pltpu.SemaphoreType.DMA,
            pltpu.SemaphoreType.DMA,
        ],
        compiler_params=pltpu.CompilerParams(
            needs_layout_passes=False, use_tc_tiling_on_sc=False),
    )
    def sc_kernel(table_hbm, i_hbm, k_hbm, out_hbm,
                  iv, kv, uiv, ukv, outv, gsem, wsem):
        wid = lax.axis_index("s") * _NC + lax.axis_index("c")
        base_b = wid * bpw
        pltpu.sync_copy(i_hbm.at[pl.ds(base_b, bpw)], iv)
        pltpu.sync_copy(k_hbm.at[wid], kv)

        def fire(s):
            buf = s % 2
            cs = [pltpu.async_copy(
                table_hbm.at[iv.at[pl.ds(s * _NB, _NB)]],
                uiv.at[buf], gsem)]
            for j in range(krows):
                cs.append(pltpu.async_copy(
                    table_hbm.at[kv.at[s * krows + j]],
                    ukv.at[buf].at[pl.ds(j * 128, 128)], gsem))
            return cs

        pend = {0: fire(0), 1: fire(1)}
        wb = {}
        for s in range(nsub):
            buf = s % 2
            for c in pend.pop(s):
                c.wait()
            if s - 2 in wb:
                wb.pop(s - 2).wait()
            ukb = ukv.at[buf]
            uib = uiv.at[buf]
            outb = outv.at[buf]

            @pl.loop(0, ngrp)
            def _(g):
                p0 = g * _L
                pair = p0 + lax.iota(jnp.int32, _L)
                brow = pair // N
                c0 = jnp.zeros((_L,), jnp.int32)
                a0 = -(plsc.load_gather(ukb, [pair, c0])
                       * plsc.load_gather(uib, [brow, c0]))
                a1 = jnp.zeros((_L,), jnp.float32)
                a2 = jnp.zeros((_L,), jnp.float32)
                a3 = jnp.zeros((_L,), jnp.float32)
                accs = [a0, a1, a2, a3]
                for d0 in range(1, D):
                    cd = jnp.full((_L,), d0, jnp.int32)
                    prod = (plsc.load_gather(ukb, [pair, cd])
                            * plsc.load_gather(uib, [brow, cd]))
                    accs[d0 % 4] = accs[d0 % 4] + prod
                outb[pl.ds(p0, _L)] = ((accs[0] + accs[1])
                                       + (accs[2] + accs[3]))

            wb[s] = pltpu.async_copy(
                outb, out_hbm.at[pl.ds((base_b + s * _NB) * N, rows)], wsem)
            if s + 2 < nsub:
                pend[s + 2] = fire(s + 2)
        for s in sorted(wb):
            wb.pop(s).wait()

    return sc_kernel(t32, idx_i, ks3d)


def _sc_pack(tab_flat, D, V, Vr):
    RL = 1920
    W = RL + 8
    W2 = W + 1
    nchunks = (Vr + RL - 1) // RL
    last_r0 = Vr - RL
    maxk = (nchunks + _NW - 1) // _NW
    mesh = plsc.VectorSubcoreMesh(core_axis_name="c", subcore_axis_name="s")

    @pl.kernel(
        out_type=jax.ShapeDtypeStruct((Vr, D), jnp.float32),
        mesh=mesh,
        scratch_types=[
            pltpu.VMEM((D, W2), jnp.float32),
            pltpu.VMEM((RL, D), jnp.float32),
            pltpu.SemaphoreType.DMA,
        ],
        compiler_params=pltpu.CompilerParams(
            needs_layout_passes=False, use_tc_tiling_on_sc=False),
    )
    def pack_kernel(tab_hbm, out_hbm, inb, outb, sem):
        wid = lax.axis_index("s") * _NC + lax.axis_index("c")
        lo_lanes = lax.iota(jnp.int32, _L)
        hi_lanes = lo_lanes + _L
        shift = lax.iota(jnp.int32, _L) % 8

        @pl.loop(0, maxk)
        def _(k):
            c = wid + k * _NW

            @pl.when(c < nchunks)
            def _():
                r0 = jnp.minimum(c * RL, last_r0)
                cps = []
                for d in range(D):
                    start = d * V + r0 - (d % 8)
                    cps.append(pltpu.async_copy(
                        tab_hbm.at[pl.ds(start, W)],
                        inb.at[d].at[pl.ds(0, W)], sem))
                for cp in cps:
                    cp.wait()

                @pl.loop(0, RL, init_carry=shift, unroll=8)
                def _(rl, colv):
                    outb[rl, pl.ds(0, _L)] = plsc.load_gather(
                        inb, [lo_lanes, colv])
                    outb[rl, pl.ds(_L, _L)] = plsc.load_gather(
                        inb, [hi_lanes, colv])
                    return colv + 1

                pltpu.sync_copy(outb, out_hbm.at[pl.ds(r0, RL)])

    return pack_kernel(tab_flat)


def _tc_tail(lsp2d):
    Bn = lsp2d.shape[0]

    def body(lsp_ref, out_ref):
        dd = -lsp_ref[...]
        dd = jnp.where(dd <= 1.0, jnp.float32(1.0 + 1e-6), dd)
        dd = -jnp.log(dd + jnp.sqrt(dd * dd - 1.0))
        lse = jnp.log(jnp.sum(jnp.exp(dd), axis=1) + 1e-6)
        out_ref[...] = lse - dd[:, 0]

    return pl.pallas_call(
        body,
        out_shape=jax.ShapeDtypeStruct((Bn,), jnp.float32),
    )(lsp2d)


def kernel(table, I, Ks):
    B, N = Ks.shape
    V, D = table.shape
    t32 = _sc_pack(jnp.transpose(table).reshape(-1), D, V, V - 1)
    ks3d = Ks.reshape(_NW, B * N // (128 * _NW), 128).astype(jnp.int32)
    lsp = _sc_lorentz_products(t32, I.astype(jnp.int32), ks3d, B, N, D)
    return _tc_tail(lsp.reshape(B, N))


# speedup vs baseline: 4.5765x; 4.5765x over previous
import jax
import jax.numpy as jnp
from jax import lax
from jax.experimental import pallas as pl
from jax.experimental.pallas import tpu as pltpu
from jax.experimental.pallas import tpu_sc as plsc

_NC = 2
_NS = 16
_L = 16
_NW = _NC * _NS

_NB = 64


def _sc_lorentz_products(t32, idx_i, ks3d, B, N, D):
    bpw = B // _NW
    nsub = bpw // _NB
    rows = _NB * N
    ngrp = rows // _L
    krows = rows // 128
    kpw = bpw * N // 128

    mesh = plsc.VectorSubcoreMesh(core_axis_name="c", subcore_axis_name="s")

    @pl.kernel(
        out_type=jax.ShapeDtypeStruct((B * N,), jnp.float32),
        mesh=mesh,
        scratch_types=[
            pltpu.VMEM((bpw,), jnp.int32),
            pltpu.VMEM((kpw, 128), jnp.int32),
            pltpu.VMEM((2, _NB, D), jnp.float32),
            pltpu.VMEM((2, rows, D), jnp.float32),
            pltpu.VMEM((2, rows), jnp.float32),
            pltpu.SemaphoreType.DMA,
            pltpu.SemaphoreType.DMA,
        ],
        compiler_params=pltpu.CompilerParams(
            needs_layout_passes=False, use_tc_tiling_on_sc=False),
    )
    def sc_kernel(table_hbm, i_hbm, k_hbm, out_hbm,
                  iv, kv, uiv, ukv, outv, gsem, wsem):
        wid = lax.axis_index("s") * _NC + lax.axis_index("c")
        base_b = wid * bpw
        pltpu.sync_copy(i_hbm.at[pl.ds(base_b, bpw)], iv)
        pltpu.sync_copy(k_hbm.at[wid], kv)

        def fire(s):
            buf = s % 2
            cs = [pltpu.async_copy(
                table_hbm.at[iv.at[pl.ds(s * _NB, _NB)]],
                uiv.at[buf], gsem)]
            for j in range(krows):
                cs.append(pltpu.async_copy(
                    table_hbm.at[kv.at[s * krows + j]],
                    ukv.at[buf].at[pl.ds(j * 128, 128)], gsem))
            return cs

        pend = {0: fire(0), 1: fire(1)}
        wb = {}
        for s in range(nsub):
            buf = s % 2
            for c in pend.pop(s):
                c.wait()
            if s - 2 in wb:
                wb.pop(s - 2).wait()
            ukb = ukv.at[buf]
            uib = uiv.at[buf]
            outb = outv.at[buf]

            @pl.loop(0, ngrp)
            def _(g):
                p0 = g * _L
                pair = p0 + lax.iota(jnp.int32, _L)
                brow = pair // N
                c0 = jnp.zeros((_L,), jnp.int32)
                a0 = -(plsc.load_gather(ukb, [pair, c0])
                       * plsc.load_gather(uib, [brow, c0]))
                a1 = jnp.zeros((_L,), jnp.float32)
                a2 = jnp.zeros((_L,), jnp.float32)
                a3 = jnp.zeros((_L,), jnp.float32)
                accs = [a0, a1, a2, a3]
                for d0 in range(1, D):
                    cd = jnp.full((_L,), d0, jnp.int32)
                    prod = (plsc.load_gather(ukb, [pair, cd])
                            * plsc.load_gather(uib, [brow, cd]))
                    accs[d0 % 4] = accs[d0 % 4] + prod
                outb[pl.ds(p0, _L)] = ((accs[0] + accs[1])
                                       + (accs[2] + accs[3]))

            wb[s] = pltpu.async_copy(
                outb, out_hbm.at[pl.ds((base_b + s * _NB) * N, rows)], wsem)
            if s + 2 < nsub:
                pend[s + 2] = fire(s + 2)
        for s in sorted(wb):
            wb.pop(s).wait()

    return sc_kernel(t32, idx_i, ks3d)


def _tc_pack(tabT):
    D, V = tabT.shape
    blocks = (V + 4095) // 4096

    def body(x_ref, o_ref):
        y = jnp.transpose(x_ref[...])
        y2 = y.reshape(1024, 4, D)
        for jj in range(4):
            o_ref[:, jj * D:(jj + 1) * D] = y2[:, jj, :]

    return pl.pallas_call(
        body,
        grid=(blocks,),
        in_specs=[pl.BlockSpec((D, 4096), lambda i: (0, i))],
        out_specs=pl.BlockSpec((1024, 4 * D), lambda i: (i, 0)),
        out_shape=jax.ShapeDtypeStruct((blocks * 1024, 4 * D), jnp.float32),
    )(tabT)


def _tc_tail(lsp2d):
    Bn = lsp2d.shape[0]

    def body(lsp_ref, out_ref):
        dd = -lsp_ref[...]
        dd = jnp.where(dd <= 1.0, jnp.float32(1.0 + 1e-6), dd)
        dd = -jnp.log(dd + jnp.sqrt(dd * dd - 1.0))
        lse = jnp.log(jnp.sum(jnp.exp(dd), axis=1) + 1e-6)
        out_ref[...] = lse - dd[:, 0]

    return pl.pallas_call(
        body,
        out_shape=jax.ShapeDtypeStruct((Bn,), jnp.float32),
    )(lsp2d)


def kernel(table, I, Ks):
    B, N = Ks.shape
    V, D = table.shape
    t128 = _tc_pack(jnp.transpose(table))
    t32 = t128.reshape(t128.shape[0] * 4, D)
    ks3d = Ks.reshape(_NW, B * N // (128 * _NW), 128).astype(jnp.int32)
    lsp = _sc_lorentz_products(t32, I.astype(jnp.int32), ks3d, B, N, D)
    return _tc_tail(lsp.reshape(B, N))
